# baseline (speedup 1.0000x reference)
import functools

import jax
import jax.numpy as jnp
from jax import lax
from jax.experimental import pallas as pl
from jax.experimental.pallas import tpu as pltpu
from jax.experimental.pallas import tpu_sc as plsc

F = 26
V = 100000
D = 32
B = 16384

NC = 2
NS = 16
NW = NC * NS
BPW = B // NW
CHUNK = 64
NCB = BPW // CHUNK


def _sc_body(tab_hbm, x_hbm, oute_hbm, outo_hbm, xblk_v, stage_v,
             acce_v, acco_v, sem):
    wid = lax.axis_index("s") * NC + lax.axis_index("c")
    base_b = wid * BPW

    pltpu.sync_copy(x_hbm.at[:, pl.ds(wid * NCB, NCB), :], xblk_v)

    def issue(cb):
        for f in range(F):
            pltpu.async_copy(tab_hbm.at[f].at[xblk_v.at[f, cb]],
                             stage_v.at[lax.rem(cb, 2), f], sem)

    def drain(p):
        pltpu.make_async_copy(tab_hbm.at[:, pl.ds(0, CHUNK), :],
                              stage_v.at[p], sem).wait()

    issue(0)

    def cb_body(cb, c):
        p = lax.rem(cb, 2)
        drain(p)

        @pl.when(cb + 1 < NCB)
        def _():
            issue(cb + 1)

        def row_body(r, c2):
            row = stage_v[p, 0, r, :]
            s_e, s_o = plsc.unpack(row, format=plsc.PackFormat.INTERLEAVED)
            for f in range(1, F):
                row = stage_v[p, f, r, :]
                a, b = plsc.unpack(row, format=plsc.PackFormat.INTERLEAVED)
                s_e = s_e + a
                s_o = s_o + b
            acce_v[cb * CHUNK + r, :] = s_e
            acco_v[cb * CHUNK + r, :] = s_o
            return c2

        lax.fori_loop(0, CHUNK, row_body, 0)
        return c

    lax.fori_loop(0, NCB, cb_body, 0)

    pltpu.sync_copy(acce_v, oute_hbm.at[pl.ds(base_b, BPW)])
    pltpu.sync_copy(acco_v, outo_hbm.at[pl.ds(base_b, BPW)])


@jax.jit
def _sc_lookup(tab_bf, x3):
    mesh = plsc.VectorSubcoreMesh(core_axis_name="c", subcore_axis_name="s",
                                  num_cores=NC, num_subcores=NS)
    return pl.kernel(
        _sc_body,
        out_type=(jax.ShapeDtypeStruct((B, D // 2), jnp.float32),
                  jax.ShapeDtypeStruct((B, D // 2), jnp.float32)),
        mesh=mesh,
        scratch_types=[
            pltpu.VMEM((F, NCB, CHUNK), jnp.int32),
            pltpu.VMEM((2, F, CHUNK, D), jnp.bfloat16),
            pltpu.VMEM((BPW, D // 2), jnp.float32),
            pltpu.VMEM((BPW, D // 2), jnp.float32),
            pltpu.SemaphoreType.DMA,
        ],
        compiler_params=pltpu.CompilerParams(use_tc_tiling_on_sc=False,
                                             needs_layout_passes=False),
    )(tab_bf, x3)


def kernel(x, tables):
    tab_bf = tables.astype(jnp.bfloat16)
    x3 = x.T.reshape(F, B // CHUNK, CHUNK)
    e, o = _sc_lookup(tab_bf, x3)
    return jnp.stack([e, o], axis=-1).reshape(B, D)


# speedup vs baseline: 1.2169x; 1.2169x over previous
import functools

import jax
import jax.numpy as jnp
from jax import lax
from jax.experimental import pallas as pl
from jax.experimental.pallas import tpu as pltpu
from jax.experimental.pallas import tpu_sc as plsc

F = 26
V = 100000
D = 32
B = 16384

NC = 2
NS = 16
NW = NC * NS
BPW = B // NW
CHUNK = 32
NCB = BPW // CHUNK


def _sc_body(tab_hbm, x_hbm, out_hbm, xblk_v, stage_v, acc_v, sem):
    wid = lax.axis_index("s") * NC + lax.axis_index("c")
    base_b = wid * BPW

    pltpu.sync_copy(x_hbm.at[:, pl.ds(wid * NCB, NCB), :], xblk_v)

    def issue(cb):
        for f in range(F):
            pltpu.async_copy(tab_hbm.at[f].at[xblk_v.at[f, cb]],
                             stage_v.at[lax.rem(cb, 2), f], sem)

    def drain(p):
        pltpu.make_async_copy(tab_hbm.at[0].at[pl.ds(0, CHUNK)],
                              stage_v.at[p], sem).wait()

    issue(0)

    def cb_body(cb, c):
        p = lax.rem(cb, 2)
        drain(p)

        @pl.when(cb + 1 < NCB)
        def _():
            issue(cb + 1)

        def row_body(r, c2):
            s_lo = stage_v[p, 0, r, pl.ds(0, 16)]
            s_hi = stage_v[p, 0, r, pl.ds(16, 16)]
            for f in range(1, F):
                s_lo = s_lo + stage_v[p, f, r, pl.ds(0, 16)]
                s_hi = s_hi + stage_v[p, f, r, pl.ds(16, 16)]
            acc_v[cb * CHUNK + r, pl.ds(0, 16)] = s_lo
            acc_v[cb * CHUNK + r, pl.ds(16, 16)] = s_hi
            return c2

        lax.fori_loop(0, CHUNK, row_body, 0)
        return c

    lax.fori_loop(0, NCB, cb_body, 0)

    pltpu.sync_copy(acc_v, out_hbm.at[pl.ds(base_b, BPW)])


@jax.jit
def _sc_lookup(tables, x3):
    mesh = plsc.VectorSubcoreMesh(core_axis_name="c", subcore_axis_name="s",
                                  num_cores=NC, num_subcores=NS)
    return pl.kernel(
        _sc_body,
        out_type=jax.ShapeDtypeStruct((B, D), jnp.float32),
        mesh=mesh,
        scratch_types=[
            pltpu.VMEM((F, NCB, CHUNK), jnp.int32),
            pltpu.VMEM((2, F, CHUNK, D), jnp.float32),
            pltpu.VMEM((BPW, D), jnp.float32),
            pltpu.SemaphoreType.DMA,
        ],
        compiler_params=pltpu.CompilerParams(use_tc_tiling_on_sc=False,
                                             needs_layout_passes=False),
    )(tables, x3)


def kernel(x, tables):
    x3 = x.T.reshape(F, B // CHUNK, CHUNK)
    return _sc_lookup(tables, x3)
